# baseline (speedup 1.0000x reference)
import functools

import jax
import jax.numpy as jnp
from jax.experimental import pallas as pl
from jax.experimental.pallas import tpu as pltpu


def _silu(x):
    return x * jax.nn.sigmoid(x)


def _fused_kernel(s_ref, sim_ref, idxm_ref, emb_ref, w1_ref, b1_ref, w2_ref,
                  b2_ref, w3_ref, b3_ref, w4_ref, b4_ref, out_ref, zbt_ref):
    pi = pl.program_id(0)

    @pl.when(pi == 0)
    def _init():
        h = _silu(jnp.dot(emb_ref[:, :], w1_ref[:, :],
                          preferred_element_type=jnp.float32) + b1_ref[:, :])
        sim_e = jnp.dot(h, w2_ref[:, :],
                        preferred_element_type=jnp.float32) + b2_ref[:, :]
        zbt_ref[:, :] = jnp.dot(sim_e, w3_ref[:, :],
                                preferred_element_type=jnp.float32) + b3_ref[:, :]
        out_ref[:, :] = jnp.zeros_like(out_ref)

    a = jnp.dot(s_ref[:, :], w3_ref[:, :], preferred_element_type=jnp.float32)

    sim_row = sim_ref[0]
    sim_iota = jax.lax.broadcasted_iota(jnp.int32, (2, 1), 0)
    ohsim_t = (sim_row == sim_iota).astype(jnp.float32)
    zb = jax.lax.dot_general(ohsim_t, zbt_ref[:, :],
                             (((0,), (0,)), ((), ())),
                             preferred_element_type=jnp.float32)
    h2 = _silu(a + zb)
    y = jnp.dot(h2, w4_ref[:, :], preferred_element_type=jnp.float32) \
        + b4_ref[:, :]

    idx_row = idxm_ref[0]
    seg_iota = jax.lax.broadcasted_iota(jnp.int32, (16, 1), 0)
    oh_t = (idx_row == seg_iota).astype(jnp.float32)
    out_ref[:, :] += jnp.dot(oh_t, y, preferred_element_type=jnp.float32)


@functools.partial(jax.jit, static_argnames=("block_rows", "interpret"))
def _run(scalar_representation, simulation_idx, idx_m, emb_table, W1, b1, W2,
         b2, W3, b3, W4, b4, block_rows=2048, interpret=False):
    n, n_in = scalar_representation.shape
    n_hidden = W3.shape[1]
    grid = n // block_rows

    sim3 = simulation_idx.astype(jnp.int32).reshape(grid, 1, block_rows)
    idxm3 = idx_m.astype(jnp.int32).reshape(grid, 1, block_rows)

    out = pl.pallas_call(
        _fused_kernel,
        grid=(grid,),
        in_specs=[
            pl.BlockSpec((block_rows, n_in), lambda i: (i, 0)),
            pl.BlockSpec((1, 1, block_rows), lambda i: (i, 0, 0)),
            pl.BlockSpec((1, 1, block_rows), lambda i: (i, 0, 0)),
            pl.BlockSpec(emb_table.shape, lambda i: (0, 0)),
            pl.BlockSpec(W1.shape, lambda i: (0, 0)),
            pl.BlockSpec((1, b1.shape[0]), lambda i: (0, 0)),
            pl.BlockSpec(W2.shape, lambda i: (0, 0)),
            pl.BlockSpec((1, b2.shape[0]), lambda i: (0, 0)),
            pl.BlockSpec(W3.shape, lambda i: (0, 0)),
            pl.BlockSpec((1, b3.shape[0]), lambda i: (0, 0)),
            pl.BlockSpec(W4.shape, lambda i: (0, 0)),
            pl.BlockSpec((1, 1), lambda i: (0, 0)),
        ],
        out_specs=pl.BlockSpec((16, 1), lambda i: (0, 0)),
        out_shape=jax.ShapeDtypeStruct((16, 1), jnp.float32),
        scratch_shapes=[pltpu.VMEM((2, n_hidden), jnp.float32)],
        interpret=interpret,
    )(scalar_representation, sim3, idxm3, emb_table, W1, b1.reshape(1, -1),
      W2, b2.reshape(1, -1), W3, b3.reshape(1, -1), W4, b4.reshape(1, 1))
    return out.reshape(16)


def kernel(scalar_representation, simulation_idx, idx_m, emb_table, W1, b1,
           W2, b2, W3, b3, W4, b4):
    return _run(scalar_representation, simulation_idx, idx_m, emb_table, W1,
                b1, W2, b2, W3, b3, W4, b4)


# speedup vs baseline: 1.0037x; 1.0037x over previous
import functools

import jax
import jax.numpy as jnp
from jax.experimental import pallas as pl
from jax.experimental.pallas import tpu as pltpu


def _silu(x):
    return x * jax.nn.sigmoid(x)


def _fused_kernel(s_ref, sim_ref, idxm_ref, emb_ref, w1_ref, b1_ref, w2_ref,
                  b2_ref, w3_ref, b3_ref, w4_ref, b4_ref, out_ref, zbt_ref):
    pi = pl.program_id(0)

    @pl.when(pi == 0)
    def _init():
        h = _silu(jnp.dot(emb_ref[:, :], w1_ref[:, :],
                          preferred_element_type=jnp.float32) + b1_ref[:, :])
        sim_e = jnp.dot(h, w2_ref[:, :],
                        preferred_element_type=jnp.float32) + b2_ref[:, :]
        zbt_ref[:, :] = jnp.dot(sim_e, w3_ref[:, :],
                                preferred_element_type=jnp.float32) + b3_ref[:, :]
        out_ref[:, :] = jnp.zeros_like(out_ref)

    a = jnp.dot(s_ref[:, :], w3_ref[:, :], preferred_element_type=jnp.float32)

    sim_row = sim_ref[0]
    sim_iota = jax.lax.broadcasted_iota(jnp.int32, (2, 1), 0)
    ohsim_t = (sim_row == sim_iota).astype(jnp.float32)
    zb = jax.lax.dot_general(ohsim_t, zbt_ref[:, :],
                             (((0,), (0,)), ((), ())),
                             preferred_element_type=jnp.float32)
    h2 = _silu(a + zb)
    y = jnp.dot(h2, w4_ref[:, :], preferred_element_type=jnp.float32) \
        + b4_ref[:, :]

    idx_row = idxm_ref[0]
    seg_iota = jax.lax.broadcasted_iota(jnp.int32, (16, 1), 0)
    oh_t = (idx_row == seg_iota).astype(jnp.float32)
    out_ref[:, :] += jnp.dot(oh_t, y, preferred_element_type=jnp.float32)


@functools.partial(jax.jit, static_argnames=("block_rows", "interpret"))
def _run(scalar_representation, simulation_idx, idx_m, emb_table, W1, b1, W2,
         b2, W3, b3, W4, b4, block_rows=8192, interpret=False):
    n, n_in = scalar_representation.shape
    n_hidden = W3.shape[1]
    grid = n // block_rows

    sim3 = simulation_idx.astype(jnp.int32).reshape(grid, 1, block_rows)
    idxm3 = idx_m.astype(jnp.int32).reshape(grid, 1, block_rows)

    out = pl.pallas_call(
        _fused_kernel,
        grid=(grid,),
        in_specs=[
            pl.BlockSpec((block_rows, n_in), lambda i: (i, 0)),
            pl.BlockSpec((1, 1, block_rows), lambda i: (i, 0, 0)),
            pl.BlockSpec((1, 1, block_rows), lambda i: (i, 0, 0)),
            pl.BlockSpec(emb_table.shape, lambda i: (0, 0)),
            pl.BlockSpec(W1.shape, lambda i: (0, 0)),
            pl.BlockSpec((1, b1.shape[0]), lambda i: (0, 0)),
            pl.BlockSpec(W2.shape, lambda i: (0, 0)),
            pl.BlockSpec((1, b2.shape[0]), lambda i: (0, 0)),
            pl.BlockSpec(W3.shape, lambda i: (0, 0)),
            pl.BlockSpec((1, b3.shape[0]), lambda i: (0, 0)),
            pl.BlockSpec(W4.shape, lambda i: (0, 0)),
            pl.BlockSpec((1, 1), lambda i: (0, 0)),
        ],
        out_specs=pl.BlockSpec((16, 1), lambda i: (0, 0)),
        out_shape=jax.ShapeDtypeStruct((16, 1), jnp.float32),
        scratch_shapes=[pltpu.VMEM((2, n_hidden), jnp.float32)],
        interpret=interpret,
    )(scalar_representation, sim3, idxm3, emb_table, W1, b1.reshape(1, -1),
      W2, b2.reshape(1, -1), W3, b3.reshape(1, -1), W4, b4.reshape(1, 1))
    return out.reshape(16)


def kernel(scalar_representation, simulation_idx, idx_m, emb_table, W1, b1,
           W2, b2, W3, b3, W4, b4):
    return _run(scalar_representation, simulation_idx, idx_m, emb_table, W1,
                b1, W2, b2, W3, b3, W4, b4)


# speedup vs baseline: 1.0612x; 1.0573x over previous
import functools

import jax
import jax.numpy as jnp
from jax.experimental import pallas as pl
from jax.experimental.pallas import tpu as pltpu


def _silu(x):
    return x * jax.nn.sigmoid(x)


def _fused_kernel(s_ref, sim_ref, idxm_ref, emb_ref, w1_ref, b1_ref, w2_ref,
                  b2_ref, w3_ref, b3_ref, w4_ref, b4_ref, out_ref, zbt_ref):
    pi = pl.program_id(0)

    @pl.when(pi == 0)
    def _init():
        h = _silu(jnp.dot(emb_ref[:, :], w1_ref[:, :],
                          preferred_element_type=jnp.float32) + b1_ref[:, :])
        sim_e = jnp.dot(h, w2_ref[:, :],
                        preferred_element_type=jnp.float32) + b2_ref[:, :]
        zbt_ref[:, :] = jnp.dot(sim_e, w3_ref[:, :],
                                preferred_element_type=jnp.float32) + b3_ref[:, :]
        out_ref[:, :] = jnp.zeros_like(out_ref)

    a = jnp.dot(s_ref[:, :], w3_ref[:, :], preferred_element_type=jnp.float32)

    sim_row = sim_ref[:].reshape(1, -1)
    sim_iota = jax.lax.broadcasted_iota(jnp.int32, (2, 1), 0)
    ohsim_t = (sim_row == sim_iota).astype(jnp.float32)
    zb = jax.lax.dot_general(ohsim_t, zbt_ref[:, :],
                             (((0,), (0,)), ((), ())),
                             preferred_element_type=jnp.float32)
    h2 = _silu(a + zb)
    y = jnp.dot(h2, w4_ref[:, :], preferred_element_type=jnp.float32) \
        + b4_ref[:, :]

    idx_row = idxm_ref[:].reshape(1, -1)
    seg_iota = jax.lax.broadcasted_iota(jnp.int32, (16, 1), 0)
    oh_t = (idx_row == seg_iota).astype(jnp.float32)
    out_ref[:, :] += jnp.dot(oh_t, y, preferred_element_type=jnp.float32)


@functools.partial(jax.jit, static_argnames=("block_rows", "interpret"))
def _run(scalar_representation, simulation_idx, idx_m, emb_table, W1, b1, W2,
         b2, W3, b3, W4, b4, block_rows=4096, interpret=False):
    n, n_in = scalar_representation.shape
    n_hidden = W3.shape[1]
    grid = n // block_rows


    out = pl.pallas_call(
        _fused_kernel,
        grid=(grid,),
        in_specs=[
            pl.BlockSpec((block_rows, n_in), lambda i: (i, 0)),
            pl.BlockSpec((block_rows,), lambda i: (i,)),
            pl.BlockSpec((block_rows,), lambda i: (i,)),
            pl.BlockSpec(emb_table.shape, lambda i: (0, 0)),
            pl.BlockSpec(W1.shape, lambda i: (0, 0)),
            pl.BlockSpec((1, b1.shape[0]), lambda i: (0, 0)),
            pl.BlockSpec(W2.shape, lambda i: (0, 0)),
            pl.BlockSpec((1, b2.shape[0]), lambda i: (0, 0)),
            pl.BlockSpec(W3.shape, lambda i: (0, 0)),
            pl.BlockSpec((1, b3.shape[0]), lambda i: (0, 0)),
            pl.BlockSpec(W4.shape, lambda i: (0, 0)),
            pl.BlockSpec((1, 1), lambda i: (0, 0)),
        ],
        out_specs=pl.BlockSpec((16, 1), lambda i: (0, 0)),
        out_shape=jax.ShapeDtypeStruct((16, 1), jnp.float32),
        scratch_shapes=[pltpu.VMEM((2, n_hidden), jnp.float32)],
        interpret=interpret,
    )(scalar_representation, simulation_idx.astype(jnp.int32),
      idx_m.astype(jnp.int32), emb_table, W1, b1.reshape(1, -1),
      W2, b2.reshape(1, -1), W3, b3.reshape(1, -1), W4, b4.reshape(1, 1))
    return out.reshape(16)


def kernel(scalar_representation, simulation_idx, idx_m, emb_table, W1, b1,
           W2, b2, W3, b3, W4, b4):
    return _run(scalar_representation, simulation_idx, idx_m, emb_table, W1,
                b1, W2, b2, W3, b3, W4, b4)
